# baseline (speedup 1.0000x reference)
import functools

import jax
import jax.numpy as jnp
from jax import lax
from jax.experimental import pallas as pl
from jax.experimental.pallas import tpu as pltpu
from jax.experimental.pallas import tpu_sc as plsc

NC = 2
NS = 16
NW = NC * NS

T = 50
B = 16384
D = 32
V = 1000000
CB = 128
JB = 4
NSTEP = T * JB
NCB = B // CB
OROWS = T * (D // 8) * NCB
RING = 5


def _body(idx_hbm, w_hbm, o2_hbm, idx_v, grow_v, stage_v, sem_i, sem_g,
          sem_o):
    cid = lax.axis_index("c")
    sid = lax.axis_index("s")
    wid = sid * NC + cid
    iota16 = lax.iota(jnp.int32, 16)
    rowvecs = [iota16 + (jg * 16) for jg in range(8)]

    for t in range(T):
        pltpu.async_copy(
            idx_hbm.at[pl.ds(t * NCB + wid * JB, JB)],
            idx_v.at[pl.ds(t * JB, JB)],
            sem_i,
        )
    pltpu.make_async_copy(
        idx_hbm.at[pl.ds(0, NSTEP)], idx_v, sem_i
    ).wait()

    def fire(s):
        pltpu.async_copy(
            w_hbm.at[idx_v.at[s]],
            grow_v.at[s % RING],
            sem_g,
        )

    for s in range(RING - 1):
        fire(s)

    def step_fn(s, carry):
        e = s - (RING - 1)
        pltpu.make_async_copy(
            w_hbm.at[pl.ds(0, CB)],
            grow_v.at[0],
            sem_g,
        ).wait()

        @pl.when(s < NSTEP)
        def _():
            fire(s)

        ge = e % RING
        je = e % JB
        sr = (e // JB) % 2

        @pl.when(jnp.logical_and(je == 0, e >= 8))
        def _():
            for _R in range(D // 8):
                pltpu.make_async_copy(
                    o2_hbm.at[pl.ds(0, JB)],
                    stage_v.at[sr, :, pl.ds(0, 1024)],
                    sem_o,
                ).wait()

        gev = jnp.full((16,), ge, jnp.int32)
        srv = jnp.full((16,), sr, jnp.int32)
        jev = jnp.full((16,), je, jnp.int32)
        for d in range(D):
            colv = (iota16 + d) & (D - 1)
            colv128 = colv * CB
            for jg in range(8):
                vals = plsc.load_gather(grow_v, [gev, rowvecs[jg], colv])
                plsc.store_scatter(
                    stage_v, [srv, jev, colv128 + rowvecs[jg]], vals
                )

        @pl.when(je == JB - 1)
        def _():
            t1 = e // JB
            for R in range(D // 8):
                pltpu.async_copy(
                    stage_v.at[sr, :, pl.ds(R * 1024, 1024)],
                    o2_hbm.at[pl.ds(t1 * 512 + R * NCB + wid * JB, JB)],
                    sem_o,
                )
        return carry

    lax.fori_loop(RING - 1, NSTEP + RING - 1, step_fn, 0)

    for _p in range(2):
        for _R in range(D // 8):
            pltpu.make_async_copy(
                o2_hbm.at[pl.ds(0, JB)],
                stage_v.at[0, :, pl.ds(0, 1024)],
                sem_o,
            ).wait()


def kernel(token_ids, weight):
    idx2d = token_ids.T.astype(jnp.int32).reshape(T * NCB, CB)
    mesh = plsc.VectorSubcoreMesh(core_axis_name="c", subcore_axis_name="s")
    o2 = pl.kernel(
        _body,
        out_type=jax.ShapeDtypeStruct((OROWS, 1024), jnp.float32),
        mesh=mesh,
        scratch_types=[
            pltpu.VMEM((NSTEP, CB), jnp.int32),
            pltpu.VMEM((RING, CB, D), jnp.float32),
            pltpu.VMEM((2, JB, 4096), jnp.float32),
            pltpu.SemaphoreType.DMA,
            pltpu.SemaphoreType.DMA,
            pltpu.SemaphoreType.DMA,
        ],
        compiler_params=pltpu.CompilerParams(
            use_tc_tiling_on_sc=False, needs_layout_passes=False
        ),
    )(idx2d, weight)
    o5 = o2.reshape(T, D // 8, NCB, 8, CB)
    return o5.transpose(2, 4, 0, 1, 3).reshape(B, T, D)


# speedup vs baseline: 1.0032x; 1.0032x over previous
import jax
import jax.numpy as jnp
from jax import lax
from jax.experimental import pallas as pl
from jax.experimental.pallas import tpu as pltpu
from jax.experimental.pallas import tpu_sc as plsc

NC = 2
NS = 16
NW = NC * NS

T = 50
B = 16384
D = 32
V = 1000000
CB = 128
JB = 4
NSTEP = T * JB
NCB = B // CB
OROWS = T * (D // 8) * NCB
RING = 4


def _body(idx_hbm, w_hbm, o2_hbm, idx_v, grow_v, stage_v, sem_i, sem_g,
          sem_o):
    cid = lax.axis_index("c")
    sid = lax.axis_index("s")
    wid = sid * NC + cid
    iota16 = lax.iota(jnp.int32, 16)
    rowvecs = [iota16 + (jg * 16) for jg in range(8)]

    for t in range(T):
        pltpu.async_copy(
            idx_hbm.at[pl.ds(t * NCB + wid * JB, JB)],
            idx_v.at[pl.ds(t * JB, JB)],
            sem_i,
        )
    pltpu.make_async_copy(
        idx_hbm.at[pl.ds(0, NSTEP)], idx_v, sem_i
    ).wait()

    def fire(s):
        pltpu.async_copy(
            w_hbm.at[idx_v.at[s]],
            grow_v.at[s % RING],
            sem_g,
        )

    for s in range(RING - 1):
        fire(s)

    def step_fn(s, carry):
        e = s - (RING - 1)
        pltpu.make_async_copy(
            w_hbm.at[pl.ds(0, CB)],
            grow_v.at[0],
            sem_g,
        ).wait()

        @pl.when(s < NSTEP)
        def _():
            fire(s)

        ge = e % RING
        je = e % JB
        sr = (e // JB) % 2

        @pl.when(jnp.logical_and(je == 0, e >= 8))
        def _():
            for _R in range(D // 8):
                pltpu.make_async_copy(
                    o2_hbm.at[pl.ds(0, JB)],
                    stage_v.at[sr, :, pl.ds(0, 1024)],
                    sem_o,
                ).wait()

        gev = jnp.full((16,), ge, jnp.int32)
        srv = jnp.full((16,), sr, jnp.int32)
        jev = jnp.full((16,), je, jnp.int32)
        for d in range(D):
            colv = (iota16 + d) & (D - 1)
            colv128 = colv * CB
            for jg in range(8):
                vals = plsc.load_gather(grow_v, [gev, rowvecs[jg], colv])
                plsc.store_scatter(
                    stage_v, [srv, jev, colv128 + rowvecs[jg]], vals
                )

        @pl.when(je == JB - 1)
        def _():
            t1 = e // JB
            for R in range(D // 8):
                pltpu.async_copy(
                    stage_v.at[sr, :, pl.ds(R * 1024, 1024)],
                    o2_hbm.at[pl.ds(t1 * 512 + R * NCB + wid * JB, JB)],
                    sem_o,
                )
        return carry

    lax.fori_loop(RING - 1, NSTEP + RING - 1, step_fn, 0)

    for _p in range(2):
        for _R in range(D // 8):
            pltpu.make_async_copy(
                o2_hbm.at[pl.ds(0, JB)],
                stage_v.at[0, :, pl.ds(0, 1024)],
                sem_o,
            ).wait()


def kernel(token_ids, weight):
    idx2d = token_ids.T.astype(jnp.int32).reshape(T * NCB, CB)
    mesh = plsc.VectorSubcoreMesh(core_axis_name="c", subcore_axis_name="s")
    o2 = pl.kernel(
        _body,
        out_type=jax.ShapeDtypeStruct((OROWS, 1024), jnp.float32),
        mesh=mesh,
        scratch_types=[
            pltpu.VMEM((NSTEP, CB), jnp.int32),
            pltpu.VMEM((RING, CB, D), jnp.float32),
            pltpu.VMEM((2, JB, 4096), jnp.float32),
            pltpu.SemaphoreType.DMA,
            pltpu.SemaphoreType.DMA,
            pltpu.SemaphoreType.DMA,
        ],
        compiler_params=pltpu.CompilerParams(
            use_tc_tiling_on_sc=False, needs_layout_passes=False
        ),
    )(idx2d, weight)
    o5 = o2.reshape(T, D // 8, NCB, 8, CB)
    return o5.transpose(2, 4, 0, 1, 3).reshape(B, T, D)
